# baseline (speedup 1.0000x reference)
import jax
import jax.numpy as jnp
from jax.experimental import pallas as pl

NUM_FREQ = 32
MAX_FREQ = 200.0
PI = 3.14159265358979323846
DSTEP = (MAX_FREQ / 2.0 - 1.0) / (NUM_FREQ - 1)

S0 = 3.141592653589793
S1 = -5.167712780049970
S2 = 2.550164039877345
S3 = -0.5992645293207921
S4 = 0.08214588661112823
C0 = 1.0
C1 = -4.934802200544679
C2 = 4.058712126416768
C3 = -1.3352627688545895
C4 = 0.2353306303588932

ROWS_PER_BLOCK = 1024


def _encode_block(x_ref, o_ref):
    xb = x_ref[...]
    x0 = xb[:, 0:1]
    x1 = xb[:, 1:2]
    x2 = xb[:, 2:3]

    lane = jax.lax.broadcasted_iota(jnp.int32, (1, 128), 1)
    fidx = (lane % NUM_FREQ).astype(jnp.float32)
    coord = (lane // NUM_FREQ) % 2
    m = 1.0 + fidx * DSTEP
    m0 = jnp.where(coord == 0, m, 0.0)
    m1 = jnp.where(coord == 1, m, 0.0)

    u = x0 * m0 + x1 * m1
    k = jnp.floor(u + 0.5)
    t = u - k
    t2 = t * t
    h = k * 0.5
    sgn = 1.0 - 4.0 * (h - jnp.floor(h))
    st = sgn * t
    sv = st * (S0 + t2 * (S1 + t2 * (S2 + t2 * (S3 + t2 * S4))))
    cv = sgn * (C0 + t2 * (C1 + t2 * (C2 + t2 * (C3 + t2 * C4))))
    o_ref[:, 0:128] = jnp.where(lane < 2 * NUM_FREQ, sv, cv)

    jj = jax.lax.broadcasted_iota(jnp.int32, (1, 8), 1) + 1
    o_ref[:, 128:136] = (x2.astype(jnp.int32) == jj).astype(jnp.float32)


def kernel(x):
    b, n, _ = x.shape
    rows = b * n
    xf = x.reshape(rows, 3)
    grid = rows // ROWS_PER_BLOCK
    out = pl.pallas_call(
        _encode_block,
        grid=(grid,),
        in_specs=[pl.BlockSpec((ROWS_PER_BLOCK, 3), lambda i: (i, 0))],
        out_specs=pl.BlockSpec((ROWS_PER_BLOCK, 136), lambda i: (i, 0)),
        out_shape=jax.ShapeDtypeStruct((rows, 136), x.dtype),
    )(xf)
    return out.reshape(b, n, 136)


# speedup vs baseline: 1.5097x; 1.5097x over previous
import jax
import jax.numpy as jnp
from jax.experimental import pallas as pl

NUM_FREQ = 32
MAX_FREQ = 200.0
DSTEP = (MAX_FREQ / 2.0 - 1.0) / (NUM_FREQ - 1)

S0 = 3.141592653589793
S1 = -5.167712780049970
S2 = 2.550164039877345
S3 = -0.5992645293207921
S4 = 0.08214588661112823
C0 = 1.0
C1 = -4.934802200544679
C2 = 4.058712126416768
C3 = -1.3352627688545895
C4 = 0.2353306303588932

BATCH_BLOCK = 128


def _encode_block(x_ref, o_ref):
    xb = x_ref[...]
    x0 = xb[:, :, 0:1]
    x1 = xb[:, :, 1:2]
    x2 = xb[:, :, 2:3]

    lane = jax.lax.broadcasted_iota(jnp.int32, (1, 1, 128), 2)
    fidx = (lane % NUM_FREQ).astype(jnp.float32)
    coord = (lane // NUM_FREQ) % 2
    m = 1.0 + fidx * DSTEP
    m0 = jnp.where(coord == 0, m, 0.0)
    m1 = jnp.where(coord == 1, m, 0.0)

    u = x0 * m0 + x1 * m1
    k = jnp.floor(u + 0.5)
    t = u - k
    t2 = t * t
    h = k * 0.5
    sgn = 1.0 - 4.0 * (h - jnp.floor(h))
    st = sgn * t
    sv = st * (S0 + t2 * (S1 + t2 * (S2 + t2 * (S3 + t2 * S4))))
    cv = sgn * (C0 + t2 * (C1 + t2 * (C2 + t2 * (C3 + t2 * C4))))
    o_ref[:, :, 0:128] = jnp.where(lane < 2 * NUM_FREQ, sv, cv)

    jj = jax.lax.broadcasted_iota(jnp.int32, (1, 1, 8), 2) + 1
    o_ref[:, :, 128:136] = (x2.astype(jnp.int32) == jj).astype(jnp.float32)


def kernel(x):
    b, n, _ = x.shape
    grid = b // BATCH_BLOCK
    return pl.pallas_call(
        _encode_block,
        grid=(grid,),
        in_specs=[pl.BlockSpec((BATCH_BLOCK, n, 3), lambda i: (i, 0, 0))],
        out_specs=pl.BlockSpec((BATCH_BLOCK, n, 136), lambda i: (i, 0, 0)),
        out_shape=jax.ShapeDtypeStruct((b, n, 136), x.dtype),
    )(x)


# speedup vs baseline: 8.4518x; 5.5984x over previous
import jax
import jax.numpy as jnp
from jax.experimental import pallas as pl

NUM_FREQ = 32
MAX_FREQ = 200.0
DSTEP = (MAX_FREQ / 2.0 - 1.0) / (NUM_FREQ - 1)

S0 = 3.14158476
S1 = -5.16724799
S2 = 2.54287433
S3 = -0.55715608
C0 = 0.99999528
C1 = -4.93412021
C2 = 4.04361757
C3 = -1.22933149

LANE_BLOCK = 2048


def _encode_block(x_ref, o_ref):
    ni = pl.program_id(1)
    x0 = x_ref[0, pl.ds(ni, 1), :]
    x1 = x_ref[1, pl.ds(ni, 1), :]
    x2 = x_ref[2, pl.ds(ni, 1), :]
    ll = x0.shape[-1]

    row = jax.lax.broadcasted_iota(jnp.int32, (64, 1), 0)
    f = 1.0 + (row % NUM_FREQ).astype(jnp.float32) * DSTEP
    xs = jnp.concatenate(
        [jnp.broadcast_to(x0, (NUM_FREQ, ll)), jnp.broadcast_to(x1, (NUM_FREQ, ll))],
        axis=0,
    )
    u = f * xs
    k = jnp.round(u)
    t = u - k
    t2 = t * t
    sbit = jnp.left_shift(jnp.bitwise_and(k.astype(jnp.int32), 1), 31)
    sv = t * (S0 + t2 * (S1 + t2 * (S2 + t2 * S3)))
    cv = C0 + t2 * (C1 + t2 * (C2 + t2 * C3))
    sv = jax.lax.bitcast_convert_type(
        jax.lax.bitcast_convert_type(sv, jnp.int32) ^ sbit, jnp.float32
    )
    cv = jax.lax.bitcast_convert_type(
        jax.lax.bitcast_convert_type(cv, jnp.int32) ^ sbit, jnp.float32
    )
    o_ref[0, 0:64, :] = sv
    o_ref[0, 64:128, :] = cv

    jj = jax.lax.broadcasted_iota(jnp.int32, (8, 1), 0) + 1
    o_ref[0, 128:136, :] = (x2.astype(jnp.int32) == jj).astype(jnp.float32)


def kernel(x):
    b, n, _ = x.shape
    xt = jnp.transpose(x, (2, 1, 0))
    y = pl.pallas_call(
        _encode_block,
        grid=(b // LANE_BLOCK, n),
        in_specs=[pl.BlockSpec((3, n, LANE_BLOCK), lambda i, ni: (0, 0, i))],
        out_specs=pl.BlockSpec((1, 136, LANE_BLOCK), lambda i, ni: (ni, 0, i)),
        out_shape=jax.ShapeDtypeStruct((n, 136, b), x.dtype),
    )(xt)
    return jnp.transpose(y, (2, 0, 1))


# speedup vs baseline: 10.9567x; 1.2964x over previous
import jax
import jax.numpy as jnp
from jax.experimental import pallas as pl

NUM_FREQ = 32
MAX_FREQ = 200.0
DSTEP = (MAX_FREQ / 2.0 - 1.0) / (NUM_FREQ - 1)

S0 = 3.14158476
S1 = -5.16724799
S2 = 2.54287433
S3 = -0.55715608
C0 = 0.99999528
C1 = -4.93412021
C2 = 4.04361757
C3 = -1.22933149

LANE_BLOCK = 4096


def _encode_block(x_ref, o_ref):
    ni = pl.program_id(1)
    x0 = x_ref[0, pl.ds(ni, 1), :]
    x1 = x_ref[1, pl.ds(ni, 1), :]
    x2 = x_ref[2, pl.ds(ni, 1), :]
    ll = x0.shape[-1]

    row = jax.lax.broadcasted_iota(jnp.int32, (64, 1), 0)
    f = 1.0 + (row % NUM_FREQ).astype(jnp.float32) * DSTEP
    xs = jnp.concatenate(
        [jnp.broadcast_to(x0, (NUM_FREQ, ll)), jnp.broadcast_to(x1, (NUM_FREQ, ll))],
        axis=0,
    )
    u = f * xs
    k = jnp.round(u)
    t = u - k
    t2 = t * t
    sbit = jnp.left_shift(jnp.bitwise_and(k.astype(jnp.int32), 1), 31)
    sv = t * (S0 + t2 * (S1 + t2 * (S2 + t2 * S3)))
    cv = C0 + t2 * (C1 + t2 * (C2 + t2 * C3))
    sv = jax.lax.bitcast_convert_type(
        jax.lax.bitcast_convert_type(sv, jnp.int32) ^ sbit, jnp.float32
    )
    cv = jax.lax.bitcast_convert_type(
        jax.lax.bitcast_convert_type(cv, jnp.int32) ^ sbit, jnp.float32
    )
    o_ref[0, 0:64, :] = sv
    o_ref[0, 64:128, :] = cv

    jj = jax.lax.broadcasted_iota(jnp.int32, (8, 1), 0) + 1
    o_ref[0, 128:136, :] = (x2.astype(jnp.int32) == jj).astype(jnp.float32)


def kernel(x):
    b, n, _ = x.shape
    xt = jnp.transpose(x, (2, 1, 0))
    y = pl.pallas_call(
        _encode_block,
        grid=(b // LANE_BLOCK, n),
        in_specs=[pl.BlockSpec((3, n, LANE_BLOCK), lambda i, ni: (0, 0, i))],
        out_specs=pl.BlockSpec((1, 136, LANE_BLOCK), lambda i, ni: (ni, 0, i)),
        out_shape=jax.ShapeDtypeStruct((n, 136, b), x.dtype),
    )(xt)
    return jnp.transpose(y, (2, 0, 1))


# speedup vs baseline: 12.9060x; 1.1779x over previous
import jax
import jax.numpy as jnp
from jax.experimental import pallas as pl

NUM_FREQ = 32
MAX_FREQ = 200.0
DSTEP = (MAX_FREQ / 2.0 - 1.0) / (NUM_FREQ - 1)

S0 = 3.14158476
S1 = -5.16724799
S2 = 2.54287433
S3 = -0.55715608
C0 = 0.99999528
C1 = -4.93412021
C2 = 4.04361757
C3 = -1.22933149

LANE_BLOCK = 8192


def _encode_block(x_ref, o_ref):
    ni = pl.program_id(1)
    x0 = x_ref[0, pl.ds(ni, 1), :]
    x1 = x_ref[1, pl.ds(ni, 1), :]
    x2 = x_ref[2, pl.ds(ni, 1), :]
    ll = x0.shape[-1]

    row = jax.lax.broadcasted_iota(jnp.int32, (64, 1), 0)
    f = 1.0 + (row % NUM_FREQ).astype(jnp.float32) * DSTEP
    xs = jnp.concatenate(
        [jnp.broadcast_to(x0, (NUM_FREQ, ll)), jnp.broadcast_to(x1, (NUM_FREQ, ll))],
        axis=0,
    )
    u = f * xs
    k = jnp.round(u)
    t = u - k
    t2 = t * t
    sbit = jnp.left_shift(jnp.bitwise_and(k.astype(jnp.int32), 1), 31)
    sv = t * (S0 + t2 * (S1 + t2 * (S2 + t2 * S3)))
    cv = C0 + t2 * (C1 + t2 * (C2 + t2 * C3))
    sv = jax.lax.bitcast_convert_type(
        jax.lax.bitcast_convert_type(sv, jnp.int32) ^ sbit, jnp.float32
    )
    cv = jax.lax.bitcast_convert_type(
        jax.lax.bitcast_convert_type(cv, jnp.int32) ^ sbit, jnp.float32
    )
    o_ref[0, 0:64, :] = sv
    o_ref[0, 64:128, :] = cv

    jj = jax.lax.broadcasted_iota(jnp.int32, (8, 1), 0) + 1
    o_ref[0, 128:136, :] = (x2.astype(jnp.int32) == jj).astype(jnp.float32)


def kernel(x):
    b, n, _ = x.shape
    xt = jnp.transpose(x, (2, 1, 0))
    y = pl.pallas_call(
        _encode_block,
        grid=(b // LANE_BLOCK, n),
        in_specs=[pl.BlockSpec((3, n, LANE_BLOCK), lambda i, ni: (0, 0, i))],
        out_specs=pl.BlockSpec((1, 136, LANE_BLOCK), lambda i, ni: (ni, 0, i)),
        out_shape=jax.ShapeDtypeStruct((n, 136, b), x.dtype),
    )(xt)
    return jnp.transpose(y, (2, 0, 1))


# speedup vs baseline: 14.1503x; 1.0964x over previous
import jax
import jax.numpy as jnp
from jax.experimental import pallas as pl

NUM_FREQ = 32
MAX_FREQ = 200.0
DSTEP = (MAX_FREQ / 2.0 - 1.0) / (NUM_FREQ - 1)

S0 = 3.14158476
S1 = -5.16724799
S2 = 2.54287433
S3 = -0.55715608
C0 = 0.99999528
C1 = -4.93412021
C2 = 4.04361757
C3 = -1.22933149

LANE_BLOCK = 16384


def _encode_block(x_ref, o_ref):
    ni = pl.program_id(1)
    x0 = x_ref[0, pl.ds(ni, 1), :]
    x1 = x_ref[1, pl.ds(ni, 1), :]
    x2 = x_ref[2, pl.ds(ni, 1), :]
    ll = x0.shape[-1]

    row = jax.lax.broadcasted_iota(jnp.int32, (64, 1), 0)
    f = 1.0 + (row % NUM_FREQ).astype(jnp.float32) * DSTEP
    xs = jnp.concatenate(
        [jnp.broadcast_to(x0, (NUM_FREQ, ll)), jnp.broadcast_to(x1, (NUM_FREQ, ll))],
        axis=0,
    )
    u = f * xs
    k = jnp.round(u)
    t = u - k
    t2 = t * t
    sbit = jnp.left_shift(jnp.bitwise_and(k.astype(jnp.int32), 1), 31)
    sv = t * (S0 + t2 * (S1 + t2 * (S2 + t2 * S3)))
    cv = C0 + t2 * (C1 + t2 * (C2 + t2 * C3))
    sv = jax.lax.bitcast_convert_type(
        jax.lax.bitcast_convert_type(sv, jnp.int32) ^ sbit, jnp.float32
    )
    cv = jax.lax.bitcast_convert_type(
        jax.lax.bitcast_convert_type(cv, jnp.int32) ^ sbit, jnp.float32
    )
    o_ref[0, 0:64, :] = sv
    o_ref[0, 64:128, :] = cv

    jj = jax.lax.broadcasted_iota(jnp.int32, (8, 1), 0) + 1
    o_ref[0, 128:136, :] = (x2.astype(jnp.int32) == jj).astype(jnp.float32)


def kernel(x):
    b, n, _ = x.shape
    xt = jnp.transpose(x, (2, 1, 0))
    y = pl.pallas_call(
        _encode_block,
        grid=(b // LANE_BLOCK, n),
        in_specs=[pl.BlockSpec((3, n, LANE_BLOCK), lambda i, ni: (0, 0, i))],
        out_specs=pl.BlockSpec((1, 136, LANE_BLOCK), lambda i, ni: (ni, 0, i)),
        out_shape=jax.ShapeDtypeStruct((n, 136, b), x.dtype),
    )(xt)
    return jnp.transpose(y, (2, 0, 1))


# speedup vs baseline: 16.0415x; 1.1337x over previous
import jax
import jax.numpy as jnp
from jax.experimental import pallas as pl

NUM_FREQ = 32
MAX_FREQ = 200.0
DSTEP = (MAX_FREQ / 2.0 - 1.0) / (NUM_FREQ - 1)

S0 = 3.14158476
S1 = -5.16724799
S2 = 2.54287433
S3 = -0.55715608
C0 = 0.99999528
C1 = -4.93412021
C2 = 4.04361757
C3 = -1.22933149

LANE_BLOCK = 16384


def _sincos(u):
    k = jnp.round(u)
    t = u - k
    t2 = t * t
    sbit = jnp.left_shift(jnp.bitwise_and(k.astype(jnp.int32), 1), 31)
    s = t * (S0 + t2 * (S1 + t2 * (S2 + t2 * S3)))
    c = C0 + t2 * (C1 + t2 * (C2 + t2 * C3))
    s = jax.lax.bitcast_convert_type(
        jax.lax.bitcast_convert_type(s, jnp.int32) ^ sbit, jnp.float32
    )
    c = jax.lax.bitcast_convert_type(
        jax.lax.bitcast_convert_type(c, jnp.int32) ^ sbit, jnp.float32
    )
    return s, c


def _encode_block(x_ref, o_ref):
    ni = pl.program_id(1)
    x0 = x_ref[0, pl.ds(ni, 1), :]
    x1 = x_ref[1, pl.ds(ni, 1), :]
    x2 = x_ref[2, pl.ds(ni, 1), :]
    ll = x0.shape[-1]

    jrow = jax.lax.broadcasted_iota(jnp.int32, (8, 1), 0).astype(jnp.float32)
    f8 = 1.0 + jrow * DSTEP

    x0b = jnp.broadcast_to(x0, (8, ll))
    x1b = jnp.broadcast_to(x1, (8, ll))
    sA0, cA0 = _sincos(x0b * f8)
    sB0, cB0 = _sincos(x1b * f8)

    x01 = jnp.concatenate([x0, x1], axis=0)
    x0101 = jnp.concatenate([x01, x01], axis=0)
    xs8 = jnp.concatenate([x0101, x0101], axis=0)
    mstep = (8.0 * DSTEP) + jnp.where(jrow // 2 == 1, 8.0 * DSTEP, 0.0)
    sS, cS = _sincos(xs8 * mstep)

    s8a = jnp.broadcast_to(sS[0:1], (8, ll))
    c8a = jnp.broadcast_to(cS[0:1], (8, ll))
    s8b = jnp.broadcast_to(sS[1:2], (8, ll))
    c8b = jnp.broadcast_to(cS[1:2], (8, ll))
    s16a = jnp.broadcast_to(sS[2:3], (8, ll))
    c16a = jnp.broadcast_to(cS[2:3], (8, ll))
    s16b = jnp.broadcast_to(sS[3:4], (8, ll))
    c16b = jnp.broadcast_to(cS[3:4], (8, ll))

    sA1 = sA0 * c8a + cA0 * s8a
    cA1 = cA0 * c8a - sA0 * s8a
    sB1 = sB0 * c8b + cB0 * s8b
    cB1 = cB0 * c8b - sB0 * s8b
    sA2 = sA0 * c16a + cA0 * s16a
    cA2 = cA0 * c16a - sA0 * s16a
    sA3 = sA1 * c16a + cA1 * s16a
    cA3 = cA1 * c16a - sA1 * s16a
    sB2 = sB0 * c16b + cB0 * s16b
    cB2 = cB0 * c16b - sB0 * s16b
    sB3 = sB1 * c16b + cB1 * s16b
    cB3 = cB1 * c16b - sB1 * s16b

    o_ref[0, 0:8, :] = sA0
    o_ref[0, 8:16, :] = sA1
    o_ref[0, 16:24, :] = sA2
    o_ref[0, 24:32, :] = sA3
    o_ref[0, 32:40, :] = sB0
    o_ref[0, 40:48, :] = sB1
    o_ref[0, 48:56, :] = sB2
    o_ref[0, 56:64, :] = sB3
    o_ref[0, 64:72, :] = cA0
    o_ref[0, 72:80, :] = cA1
    o_ref[0, 80:88, :] = cA2
    o_ref[0, 88:96, :] = cA3
    o_ref[0, 96:104, :] = cB0
    o_ref[0, 104:112, :] = cB1
    o_ref[0, 112:120, :] = cB2
    o_ref[0, 120:128, :] = cB3

    jj = jax.lax.broadcasted_iota(jnp.int32, (8, 1), 0) + 1
    o_ref[0, 128:136, :] = (x2.astype(jnp.int32) == jj).astype(jnp.float32)


def kernel(x):
    b, n, _ = x.shape
    xt = jnp.transpose(x, (2, 1, 0))
    y = pl.pallas_call(
        _encode_block,
        grid=(b // LANE_BLOCK, n),
        in_specs=[pl.BlockSpec((3, n, LANE_BLOCK), lambda i, ni: (0, 0, i))],
        out_specs=pl.BlockSpec((1, 136, LANE_BLOCK), lambda i, ni: (ni, 0, i)),
        out_shape=jax.ShapeDtypeStruct((n, 136, b), x.dtype),
    )(xt)
    return jnp.transpose(y, (2, 0, 1))
